# baseline (speedup 1.0000x reference)
import jax
import jax.numpy as jnp
from jax import lax
from jax.experimental import pallas as pl
from jax.experimental.pallas import tpu as pltpu

B, C, H, W = 16, 512, 32, 32
HW = H * W
EMBED_DIM, TEXT_DIM, NUM_TOPK = 256, 768, 100


def _row_mask(d8):
    keys = lax.bitcast_convert_type(d8, jnp.int32)
    k_count = jnp.int32(NUM_TOPK)

    def val_step(_, carry):
        lo, hi = carry
        mid = lo + (hi - lo) // 2
        cnt = jnp.sum((keys > mid).astype(jnp.int32))
        pred = cnt < k_count
        return jnp.where(pred, lo, mid), jnp.where(pred, mid, hi)

    _, t_val = lax.fori_loop(
        0, 32, val_step,
        (jnp.int32(-1), jnp.int32(jnp.iinfo(jnp.int32).max - 1)))

    n_gt = jnp.sum((keys > t_val).astype(jnp.int32))
    need = k_count - n_gt
    eq = keys == t_val
    idx = lax.broadcasted_iota(jnp.int32, (8, 128), 0) * 128 + \
        lax.broadcasted_iota(jnp.int32, (8, 128), 1)

    def idx_step(_, carry):
        lo, hi = carry
        mid = lo + (hi - lo) // 2
        cnt = jnp.sum((eq & (idx < mid)).astype(jnp.int32))
        pred = cnt >= need
        return jnp.where(pred, lo, mid), jnp.where(pred, mid, hi)

    _, x_star = lax.fori_loop(0, 11, idx_step,
                              (jnp.int32(0), jnp.int32(HW)))

    mask = (keys > t_val) | (eq & (idx < x_star))
    return mask.astype(jnp.float32)


def _body(feat_ref, wd_ref, bd_ref, wsp_ref, wout_ref, wtext_ref, temb_ref,
          bsp_ref, btext_ref, bout_ref, out_ref):
    f = feat_ref[0]
    lg = jnp.dot(wd_ref[...], f,
                 preferred_element_type=jnp.float32) + bd_ref[...]
    dens8 = jax.nn.sigmoid(lg).reshape(8, 128)
    m = _row_mask(dens8).reshape(1, HW)

    tcol = (jnp.dot(wtext_ref[...], temb_ref[...],
                    preferred_element_type=jnp.float32)
            + btext_ref[...] + bsp_ref[...])
    z1 = jnp.dot(wsp_ref[...].astype(jnp.bfloat16), f.astype(jnp.bfloat16),
                 preferred_element_type=jnp.float32) + tcol
    z2 = jnp.dot(wout_ref[...].astype(jnp.bfloat16), z1.astype(jnp.bfloat16),
                 preferred_element_type=jnp.float32) + bout_ref[...]
    out_ref[0] = jnp.where(m > 0.0, z2, f)


@jax.jit
def kernel(feat, text_emb, Wd, bd, W_sp, b_sp, W_text, b_text, W_out, b_out):
    b, c, h, w = feat.shape
    feat3 = feat.reshape(b, c, h * w)

    out = pl.pallas_call(
        _body,
        grid=(b,),
        in_specs=[
            pl.BlockSpec((1, c, h * w), lambda i: (i, 0, 0)),
            pl.BlockSpec((1, c), lambda i: (0, 0)),
            pl.BlockSpec((1, 1), lambda i: (0, 0)),
            pl.BlockSpec((EMBED_DIM, c), lambda i: (0, 0)),
            pl.BlockSpec((c, EMBED_DIM), lambda i: (0, 0)),
            pl.BlockSpec((EMBED_DIM, TEXT_DIM), lambda i: (0, 0)),
            pl.BlockSpec((TEXT_DIM, 1), lambda i: (0, 0)),
            pl.BlockSpec((EMBED_DIM, 1), lambda i: (0, 0)),
            pl.BlockSpec((EMBED_DIM, 1), lambda i: (0, 0)),
            pl.BlockSpec((c, 1), lambda i: (0, 0)),
        ],
        out_specs=pl.BlockSpec((1, c, h * w), lambda i: (i, 0, 0)),
        out_shape=jax.ShapeDtypeStruct((b, c, h * w), jnp.float32),
        compiler_params=pltpu.CompilerParams(
            dimension_semantics=("arbitrary",)),
    )(feat3, Wd.reshape(1, c), bd.reshape(1, 1), W_sp, W_out, W_text,
      text_emb.reshape(TEXT_DIM, 1), b_sp.reshape(EMBED_DIM, 1),
      b_text.reshape(EMBED_DIM, 1), b_out.reshape(c, 1))

    return out.reshape(b, c, h, w)


# speedup vs baseline: 1.4967x; 1.4967x over previous
import jax
import jax.numpy as jnp
from jax import lax
from jax.experimental import pallas as pl
from jax.experimental.pallas import tpu as pltpu

B, C, H, W = 16, 512, 32, 32
HW = H * W
EMBED_DIM, TEXT_DIM, NUM_TOPK = 256, 768, 100
G = 4
RING = 2 * G


def _compute_mask(d):
    keys = lax.bitcast_convert_type(d, jnp.int32)
    k_count = jnp.int32(NUM_TOPK)

    def val_step(_, carry):
        lo, hi = carry
        mid = lo + (hi - lo) // 2
        cnt = jnp.sum((keys > mid).astype(jnp.int32), axis=1, keepdims=True)
        pred = cnt < k_count
        return jnp.where(pred, lo, mid), jnp.where(pred, mid, hi)

    lo0 = jnp.full((G, 1), -1, jnp.int32)
    hi0 = jnp.full((G, 1), jnp.iinfo(jnp.int32).max - 1, jnp.int32)
    _, t_val = lax.fori_loop(0, 32, val_step, (lo0, hi0))

    n_gt = jnp.sum((keys > t_val).astype(jnp.int32), axis=1, keepdims=True)
    need = k_count - n_gt
    eq = keys == t_val
    idx = lax.broadcasted_iota(jnp.int32, (G, HW), 1)

    def idx_step(_, carry):
        lo, hi = carry
        mid = lo + (hi - lo) // 2
        cnt = jnp.sum((eq & (idx < mid)).astype(jnp.int32), axis=1,
                      keepdims=True)
        pred = cnt >= need
        return jnp.where(pred, lo, mid), jnp.where(pred, mid, hi)

    lo0 = jnp.zeros((G, 1), jnp.int32)
    hi0 = jnp.full((G, 1), HW, jnp.int32)
    _, x_star = lax.fori_loop(0, 11, idx_step, (lo0, hi0))

    mask = (keys > t_val) | (eq & (idx < x_star))
    return mask.astype(jnp.float32)


def _body(feat_ref, wd_ref, bd_ref, wsp_ref, wout_ref, wtext_ref, temb_ref,
          bsp_ref, btext_ref, bout_ref, out_ref, stash_ref, dens_ref,
          mask_ref):
    s = pl.program_id(0)

    @pl.when(s < B)
    def _():
        slot = lax.rem(s, RING)
        f = feat_ref[0]
        stash_ref[pl.ds(slot, 1)] = feat_ref[...]
        lg = jnp.dot(wd_ref[...], f,
                     preferred_element_type=jnp.float32) + bd_ref[...]
        dens_ref[pl.ds(slot, 1)] = jax.nn.sigmoid(lg)[None]

    @pl.when(jnp.logical_and(s < B, lax.rem(s, G) == G - 1))
    def _():
        half = lax.rem(s - (G - 1), RING)
        d = dens_ref[pl.ds(half, G)].reshape(G, HW)
        mask_ref[pl.ds(half, G)] = _compute_mask(d).reshape(G, 1, HW)

    @pl.when(s >= G)
    def _():
        slot = lax.rem(s - G, RING)
        f = stash_ref[pl.ds(slot, 1)][0]
        m = mask_ref[pl.ds(slot, 1)][0]
        tcol = (jnp.dot(wtext_ref[...], temb_ref[...],
                        preferred_element_type=jnp.float32)
                + btext_ref[...] + bsp_ref[...])
        z1 = jnp.dot(wsp_ref[...].astype(jnp.bfloat16),
                     f.astype(jnp.bfloat16),
                     preferred_element_type=jnp.float32) + tcol
        z2 = jnp.dot(wout_ref[...].astype(jnp.bfloat16),
                     z1.astype(jnp.bfloat16),
                     preferred_element_type=jnp.float32) + bout_ref[...]
        out_ref[0] = jnp.where(m > 0.0, z2, f)


@jax.jit
def kernel(feat, text_emb, Wd, bd, W_sp, b_sp, W_text, b_text, W_out, b_out):
    b, c, h, w = feat.shape
    feat3 = feat.reshape(b, c, h * w)

    out = pl.pallas_call(
        _body,
        grid=(b + G,),
        in_specs=[
            pl.BlockSpec((1, c, h * w), lambda i: (jnp.minimum(i, B - 1), 0, 0)),
            pl.BlockSpec((1, c), lambda i: (0, 0)),
            pl.BlockSpec((1, 1), lambda i: (0, 0)),
            pl.BlockSpec((EMBED_DIM, c), lambda i: (0, 0)),
            pl.BlockSpec((c, EMBED_DIM), lambda i: (0, 0)),
            pl.BlockSpec((EMBED_DIM, TEXT_DIM), lambda i: (0, 0)),
            pl.BlockSpec((TEXT_DIM, 1), lambda i: (0, 0)),
            pl.BlockSpec((EMBED_DIM, 1), lambda i: (0, 0)),
            pl.BlockSpec((EMBED_DIM, 1), lambda i: (0, 0)),
            pl.BlockSpec((c, 1), lambda i: (0, 0)),
        ],
        out_specs=pl.BlockSpec((1, c, h * w),
                               lambda i: (jnp.maximum(i - G, 0), 0, 0)),
        out_shape=jax.ShapeDtypeStruct((b, c, h * w), jnp.float32),
        scratch_shapes=[
            pltpu.VMEM((RING, C, HW), jnp.float32),
            pltpu.VMEM((RING, 1, HW), jnp.float32),
            pltpu.VMEM((RING, 1, HW), jnp.float32),
        ],
        compiler_params=pltpu.CompilerParams(
            dimension_semantics=("arbitrary",)),
    )(feat3, Wd.reshape(1, c), bd.reshape(1, 1), W_sp, W_out, W_text,
      text_emb.reshape(TEXT_DIM, 1), b_sp.reshape(EMBED_DIM, 1),
      b_text.reshape(EMBED_DIM, 1), b_out.reshape(c, 1))

    return out.reshape(b, c, h, w)


# speedup vs baseline: 1.5922x; 1.0638x over previous
import functools

import jax
import jax.numpy as jnp
from jax import lax
from jax.experimental import pallas as pl
from jax.experimental.pallas import tpu as pltpu

B, C, H, W = 16, 512, 32, 32
HW = H * W
EMBED_DIM, TEXT_DIM, NUM_TOPK = 256, 768, 100


def _compute_mask(d):
    keys = lax.bitcast_convert_type(d, jnp.int32)
    k_count = jnp.int32(NUM_TOPK)

    def val_step(_, carry):
        lo, hi = carry
        mid = lo + (hi - lo) // 2
        cnt = jnp.sum((keys > mid).astype(jnp.int32), axis=1, keepdims=True)
        pred = cnt < k_count
        return jnp.where(pred, lo, mid), jnp.where(pred, mid, hi)

    lo0 = jnp.full((B, 1), -1, jnp.int32)
    hi0 = jnp.full((B, 1), jnp.iinfo(jnp.int32).max - 1, jnp.int32)
    _, t_val = lax.fori_loop(0, 32, val_step, (lo0, hi0))

    n_gt = jnp.sum((keys > t_val).astype(jnp.int32), axis=1, keepdims=True)
    need = k_count - n_gt
    eq = keys == t_val
    idx = lax.broadcasted_iota(jnp.int32, (B, HW), 1)

    def idx_step(_, carry):
        lo, hi = carry
        mid = lo + (hi - lo) // 2
        cnt = jnp.sum((eq & (idx < mid)).astype(jnp.int32), axis=1,
                      keepdims=True)
        pred = cnt >= need
        return jnp.where(pred, lo, mid), jnp.where(pred, mid, hi)

    lo0 = jnp.zeros((B, 1), jnp.int32)
    hi0 = jnp.full((B, 1), HW, jnp.int32)
    _, x_star = lax.fori_loop(0, 11, idx_step, (lo0, hi0))

    mask = (keys > t_val) | (eq & (idx < x_star))
    return mask.astype(jnp.float32)


def _body(feat_a, feat_b, wd_ref, bd_ref, wsp_ref, wout_ref, wtext_ref,
          temb_ref, bsp_ref, btext_ref, bout_ref, out_ref, stash_ref,
          dens_ref, mask_ref):
    i = pl.program_id(0)

    def _ingest(fref):
        f = fref[0]
        stash_ref[pl.ds(i, 1)] = fref[...]
        lg = jnp.dot(wd_ref[...], f,
                     preferred_element_type=jnp.float32) + bd_ref[...]
        dens_ref[pl.ds(i, 1)] = jax.nn.sigmoid(lg)

    @pl.when(jnp.logical_and(i < B, lax.rem(i, 2) == 0))
    def _():
        _ingest(feat_a)

    @pl.when(jnp.logical_and(i < B, lax.rem(i, 2) == 1))
    def _():
        _ingest(feat_b)

    @pl.when(i == B)
    def _():
        mask_ref[...] = _compute_mask(dens_ref[...])

    @pl.when(i >= B)
    def _():
        b = i - B
        f = stash_ref[pl.ds(b, 1)][0]
        m = mask_ref[pl.ds(b, 1)]
        tcol = (jnp.dot(wtext_ref[...], temb_ref[...],
                        preferred_element_type=jnp.float32)
                + btext_ref[...] + bsp_ref[...])
        z1 = jnp.dot(wsp_ref[...].astype(jnp.bfloat16),
                     f.astype(jnp.bfloat16),
                     preferred_element_type=jnp.float32) + tcol
        z2 = jnp.dot(wout_ref[...].astype(jnp.bfloat16),
                     z1.astype(jnp.bfloat16),
                     preferred_element_type=jnp.float32) + bout_ref[...]
        out_ref[0] = jnp.where(m > 0.0, z2, f)


@jax.jit
def kernel(feat, text_emb, Wd, bd, W_sp, b_sp, W_text, b_text, W_out, b_out):
    b, c, h, w = feat.shape
    feat3 = feat.reshape(b, c, h * w)

    out = pl.pallas_call(
        _body,
        grid=(2 * b,),
        in_specs=[
            pl.BlockSpec((1, c, h * w),
                         lambda i: (jnp.bitwise_and(jnp.minimum(i, B - 1), -2),
                                    0, 0)),
            pl.BlockSpec((1, c, h * w),
                         lambda i: (jnp.minimum(jnp.bitwise_or(i, 1), B - 1),
                                    0, 0)),
            pl.BlockSpec((1, c), lambda i: (0, 0)),
            pl.BlockSpec((1, 1), lambda i: (0, 0)),
            pl.BlockSpec((EMBED_DIM, c), lambda i: (0, 0)),
            pl.BlockSpec((c, EMBED_DIM), lambda i: (0, 0)),
            pl.BlockSpec((EMBED_DIM, TEXT_DIM), lambda i: (0, 0)),
            pl.BlockSpec((TEXT_DIM, 1), lambda i: (0, 0)),
            pl.BlockSpec((EMBED_DIM, 1), lambda i: (0, 0)),
            pl.BlockSpec((EMBED_DIM, 1), lambda i: (0, 0)),
            pl.BlockSpec((c, 1), lambda i: (0, 0)),
        ],
        out_specs=pl.BlockSpec((1, c, h * w),
                               lambda i: (jnp.maximum(i - B, 0), 0, 0)),
        out_shape=jax.ShapeDtypeStruct((b, c, h * w), jnp.float32),
        scratch_shapes=[
            pltpu.VMEM((B, C, HW), jnp.float32),
            pltpu.VMEM((B, HW), jnp.float32),
            pltpu.VMEM((B, HW), jnp.float32),
        ],
        compiler_params=pltpu.CompilerParams(
            dimension_semantics=("arbitrary",)),
    )(feat3, feat3, Wd.reshape(1, c), bd.reshape(1, 1), W_sp, W_out, W_text,
      text_emb.reshape(TEXT_DIM, 1), b_sp.reshape(EMBED_DIM, 1),
      b_text.reshape(EMBED_DIM, 1), b_out.reshape(c, 1))

    return out.reshape(b, c, h, w)
